# Initial kernel scaffold; baseline (speedup 1.0000x reference)
import jax
import jax.numpy as jnp
from jax.experimental import pallas as pl


def kernel(indices, w_es):
    raise NotImplementedError("write your pallas kernel here")


# speedup vs baseline: 188.7663x; 188.7663x over previous
import functools

import jax
import jax.numpy as jnp
from jax import lax
from jax.experimental import pallas as pl
from jax.experimental.pallas import tpu as pltpu
from jax.experimental.pallas import tpu_sc as plsc

B = 4096
V = 1000
VPAD = 1008
NWORKERS = 32
ROWS_PER_WORKER = B // NWORKERS
BLK = 16
NBLK = ROWS_PER_WORKER // BLK


def _body(idx_hbm, w_hbm, avg_hbm, wt_hbm, idx_v, w_tab, acc, wt_v, sem):
    wid = lax.axis_index("s") * 2 + lax.axis_index("c")
    pltpu.sync_copy(w_hbm, w_tab)
    lanes = lax.iota(jnp.int32, 16)
    zeros16 = jnp.zeros((16,), jnp.float32)

    def block(blk, carry):
        b0 = wid * ROWS_PER_WORKER + blk * BLK
        cp = pltpu.async_copy(idx_hbm.at[pl.ds(b0, BLK), :], idx_v, sem)

        def zrow(k, c):
            for r in range(BLK):
                acc[r, pl.ds(k * 16, 16)] = zeros16
            return c
        lax.fori_loop(0, VPAD // 16, zrow, 0)
        cp.wait()

        def col(j, c):
            for u in range(8):
                i = j * 8 + u
                vi = jnp.full((16,), i, jnp.int32)
                colv = plsc.load_gather(idx_v, [lanes, vi])
                w = plsc.load_gather(w_tab, [colv])
                plsc.addupdate_scatter(acc, [lanes, colv], w)
                plsc.store_scatter(wt_v, [vi, lanes], w)
            return c
        lax.fori_loop(0, V // 8, col, 0)

        pltpu.sync_copy(acc.at[:, pl.ds(0, V)], avg_hbm.at[pl.ds(b0, BLK), :])
        pltpu.sync_copy(wt_v, wt_hbm.at[:, pl.ds(b0, BLK)])
        return carry

    lax.fori_loop(0, NBLK, block, 0)


@jax.jit
def kernel(indices, w_es):
    run = pl.kernel(
        _body,
        out_type=(
            jax.ShapeDtypeStruct((B, V), jnp.float32),
            jax.ShapeDtypeStruct((V, B), jnp.float32),
        ),
        mesh=plsc.VectorSubcoreMesh(
            core_axis_name="c", subcore_axis_name="s",
            num_cores=2, num_subcores=16,
        ),
        scratch_types=[
            pltpu.VMEM((BLK, V), jnp.int32),
            pltpu.VMEM((V,), jnp.float32),
            pltpu.VMEM((BLK, VPAD), jnp.float32),
            pltpu.VMEM((V, BLK), jnp.float32),
            pltpu.SemaphoreType.DMA,
        ],
        compiler_params=pltpu.CompilerParams(
            use_tc_tiling_on_sc=False, needs_layout_passes=False),
    )
    return run(indices.astype(jnp.int32), w_es)


# speedup vs baseline: 284.2502x; 1.5058x over previous
import functools

import jax
import jax.numpy as jnp
from jax import lax
from jax.experimental import pallas as pl
from jax.experimental.pallas import tpu as pltpu
from jax.experimental.pallas import tpu_sc as plsc

B = 4096
V = 1000
VPAD = 1008
NWORKERS = 32
ROWS_PER_WORKER = B // NWORKERS
BLK = 16
NBLK = ROWS_PER_WORKER // BLK


def _body(idx_hbm, w_hbm, avg_hbm, wt_hbm, idx_v, w_tab, acc, wt_v, sem):
    wid = lax.axis_index("s") * 2 + lax.axis_index("c")
    pltpu.sync_copy(w_hbm, w_tab)
    lanes = lax.iota(jnp.int32, 16)
    zeros16 = jnp.zeros((16,), jnp.float32)

    def block(blk, carry):
        b0 = wid * ROWS_PER_WORKER + blk * BLK
        cp = pltpu.async_copy(idx_hbm.at[pl.ds(b0, BLK), :], idx_v, sem)

        @plsc.parallel_loop(0, VPAD // 16, unroll=4)
        def _zero(k):
            for r in range(BLK):
                acc[r, pl.ds(k * 16, 16)] = zeros16
        cp.wait()

        @plsc.parallel_loop(0, V, unroll=8)
        def _col(i):
            vi = jnp.full((16,), i, jnp.int32)
            colv = plsc.load_gather(idx_v, [lanes, vi])
            w = plsc.load_gather(w_tab, [colv])
            plsc.addupdate_scatter(acc, [lanes, colv], w)
            plsc.store_scatter(wt_v, [vi, lanes], w)

        pltpu.sync_copy(acc.at[:, pl.ds(0, V)], avg_hbm.at[pl.ds(b0, BLK), :])
        pltpu.sync_copy(wt_v, wt_hbm.at[:, pl.ds(b0, BLK)])
        return carry

    lax.fori_loop(0, NBLK, block, 0)


@jax.jit
def kernel(indices, w_es):
    run = pl.kernel(
        _body,
        out_type=(
            jax.ShapeDtypeStruct((B, V), jnp.float32),
            jax.ShapeDtypeStruct((V, B), jnp.float32),
        ),
        mesh=plsc.VectorSubcoreMesh(
            core_axis_name="c", subcore_axis_name="s",
            num_cores=2, num_subcores=16,
        ),
        scratch_types=[
            pltpu.VMEM((BLK, V), jnp.int32),
            pltpu.VMEM((V,), jnp.float32),
            pltpu.VMEM((BLK, VPAD), jnp.float32),
            pltpu.VMEM((V, BLK), jnp.float32),
            pltpu.SemaphoreType.DMA,
        ],
        compiler_params=pltpu.CompilerParams(
            use_tc_tiling_on_sc=False, needs_layout_passes=False),
    )
    return run(indices.astype(jnp.int32), w_es)


# speedup vs baseline: 296.6599x; 1.0437x over previous
import functools

import jax
import jax.numpy as jnp
from jax import lax
from jax.experimental import pallas as pl
from jax.experimental.pallas import tpu as pltpu
from jax.experimental.pallas import tpu_sc as plsc

B = 4096
V = 1000
CB = 128
RB = 8
NCHUNK = V // RB
NRW = 8
LOOK = 4


def _body(idx_hbm, w_hbm, avg_hbm, wt_hbm, w_tab, semw_i, semw_o, sema_i):
    cid = lax.axis_index("c")
    sid = lax.axis_index("s")
    wid = sid * 2 + cid
    c0 = wid * CB
    lanes = lax.iota(jnp.int32, 16)
    zeros16 = jnp.zeros((16,), jnp.float32)

    pltpu.sync_copy(w_hbm, w_tab)

    def fetch(ring, sems, chunk, buf):
        pltpu.async_copy(
            idx_hbm.at[pl.ds(chunk * RB, RB), pl.ds(c0, CB)],
            ring.at[buf], sems.at[buf])

    def wait_in(ring, sems, buf):
        pltpu.make_async_copy(
            idx_hbm.at[pl.ds(0, RB), pl.ds(c0, CB)],
            ring.at[buf], sems.at[buf]).wait()

    def phase_w(ring, ring_o):
        def put(chunk, buf):
            pltpu.async_copy(
                ring_o.at[buf],
                wt_hbm.at[pl.ds(chunk * RB, RB), pl.ds(c0, CB)],
                semw_o.at[buf])

        def wait_out(buf):
            pltpu.make_async_copy(
                ring_o.at[buf],
                wt_hbm.at[pl.ds(0, RB), pl.ds(c0, CB)],
                semw_o.at[buf]).wait()

        def step(chunk, buf, do_wait_out, do_fetch):
            wait_in(ring, semw_i, buf)
            if do_wait_out:
                wait_out(buf)
            for r in range(RB):
                for u in range(CB // 16):
                    colv = ring[buf, r, pl.ds(u * 16, 16)]
                    w = plsc.load_gather(w_tab, [colv])
                    ring_o[buf, r, pl.ds(u * 16, 16)] = w
            put(chunk, buf)
            if do_fetch:
                fetch(ring, semw_i, chunk + LOOK, (buf + LOOK) % NRW)

        for b in range(LOOK):
            fetch(ring, semw_i, b, b)
        for m in range(NRW):
            step(m, m, False, True)
        def grp(j, carry):
            for b in range(NRW):
                step(j * NRW + b, b, True, True)
            return carry
        lax.fori_loop(1, NCHUNK // NRW, grp, 0)
        step(120, 0, True, True)
        step(121, 1, True, False)
        step(122, 2, True, False)
        step(123, 3, True, False)
        step(124, 4, True, False)
        for b in (5, 6, 7, 0, 1, 2, 3, 4):
            wait_out(b)

    pl.run_scoped(
        phase_w,
        pltpu.VMEM((NRW, RB, CB), jnp.int32),
        pltpu.VMEM((NRW, RB, CB), jnp.float32),
    )

    def phase_a(acc, ring):
        fetch(ring, sema_i, 0, 0)
        fetch(ring, sema_i, 1, 1)

        @plsc.parallel_loop(0, V, unroll=4)
        def _zero(row):
            for u in range(CB // 16):
                acc[row, pl.ds(u * 16, 16)] = zeros16

        def step(chunk, buf, do_fetch):
            wait_in(ring, sema_i, buf)
            for r in range(RB):
                for u in range(CB // 16):
                    cvec = jnp.int32(u * 16) + lanes
                    colv = ring[buf, r, pl.ds(u * 16, 16)]
                    w = plsc.load_gather(w_tab, [colv])
                    plsc.addupdate_scatter(acc, [colv, cvec], w)
            if do_fetch:
                fetch(ring, sema_i, chunk + 2, buf)

        def pair(j, carry):
            step(2 * j, 0, True)
            step(2 * j + 1, 1, True)
            return carry
        lax.fori_loop(0, 61, pair, 0)
        step(122, 0, True)
        step(123, 1, False)
        step(124, 0, False)

        pltpu.sync_copy(acc, avg_hbm.at[pl.ds(0, V), pl.ds(c0, CB)])

    pl.run_scoped(
        phase_a,
        pltpu.VMEM((V, CB), jnp.float32),
        pltpu.VMEM((2, RB, CB), jnp.int32),
    )


@jax.jit
def kernel(indices, w_es):
    run = pl.kernel(
        _body,
        out_type=(
            jax.ShapeDtypeStruct((V, B), jnp.float32),
            jax.ShapeDtypeStruct((V, B), jnp.float32),
        ),
        mesh=plsc.VectorSubcoreMesh(
            core_axis_name="c", subcore_axis_name="s",
            num_cores=2, num_subcores=16,
        ),
        scratch_types=[
            pltpu.VMEM((V,), jnp.float32),
            pltpu.SemaphoreType.DMA((NRW,)),
            pltpu.SemaphoreType.DMA((NRW,)),
            pltpu.SemaphoreType.DMA((2,)),
        ],
        compiler_params=pltpu.CompilerParams(
            use_tc_tiling_on_sc=True, needs_layout_passes=False),
    )
    idx_t = jnp.transpose(indices.astype(jnp.int32))
    avg_t, wt = run(idx_t, w_es)
    return jnp.transpose(avg_t), wt


# speedup vs baseline: 366.6314x; 1.2359x over previous
import functools

import jax
import jax.numpy as jnp
from jax import lax
from jax.experimental import pallas as pl
from jax.experimental.pallas import tpu as pltpu
from jax.experimental.pallas import tpu_sc as plsc

B = 4096
V = 1000
CB = 128
RB = 8
NCHUNK = V // RB
NR = 8
LOOK = 4
VH = 504


def _body(idx_hbm, w_hbm, avg_hbm, wt_hbm, w_tab, sem_i, sem_o):
    cid = lax.axis_index("c")
    sid = lax.axis_index("s")
    wid = sid * 2 + cid
    c0 = wid * CB
    lanes = lax.iota(jnp.int32, 16)
    zeros16 = jnp.zeros((16,), jnp.float32)

    pltpu.sync_copy(w_hbm, w_tab)

    def fetch(ring, chunk, buf):
        pltpu.async_copy(
            idx_hbm.at[pl.ds(chunk * RB, RB), pl.ds(c0, CB)],
            ring.at[buf], sem_i.at[buf])

    def wait_in(ring, buf):
        pltpu.make_async_copy(
            idx_hbm.at[pl.ds(0, RB), pl.ds(c0, CB)],
            ring.at[buf], sem_i.at[buf]).wait()

    def phase_w(ring, ring_o):
        def put(chunk, buf):
            pltpu.async_copy(
                ring_o.at[buf],
                wt_hbm.at[pl.ds(chunk * RB, RB), pl.ds(c0, CB)],
                sem_o.at[buf])

        def wait_out(buf):
            pltpu.make_async_copy(
                ring_o.at[buf],
                wt_hbm.at[pl.ds(0, RB), pl.ds(c0, CB)],
                sem_o.at[buf]).wait()

        def step(chunk, buf):
            @pl.when(chunk < NCHUNK)
            def _():
                wait_in(ring, buf)

                @pl.when(chunk >= NR)
                def _():
                    wait_out(buf)

                @plsc.parallel_loop(0, RB, unroll=2)
                def _rows(r):
                    for u in range(CB // 16):
                        colv = ring[buf, r, pl.ds(u * 16, 16)]
                        w = plsc.load_gather(w_tab, [colv])
                        ring_o[buf, r, pl.ds(u * 16, 16)] = w

                put(chunk, buf)

                @pl.when(chunk + LOOK < NCHUNK)
                def _():
                    fetch(ring, chunk + LOOK, (buf + LOOK) % NR)

        for b in range(LOOK):
            fetch(ring, b, b)

        def grp(j, carry):
            for b in range(NR):
                step(j * NR + b, b)
            return carry
        lax.fori_loop(0, (NCHUNK + NR - 1) // NR, grp, 0)
        for b in range(NR):
            wait_out(b)

    pl.run_scoped(
        phase_w,
        pltpu.VMEM((NR, RB, CB), jnp.int32),
        pltpu.VMEM((NR, RB, CB), jnp.float32),
    )

    def phase_a(acc, ring):
        def walk(h, carry):
            v0 = h * (V - VH)

            for b in range(LOOK):
                fetch(ring, b, b)

            @plsc.parallel_loop(0, VH, unroll=4)
            def _zero(row):
                for u in range(CB // 16):
                    acc[row, pl.ds(u * 16, 16)] = zeros16

            def step(chunk, buf):
                @pl.when(chunk < NCHUNK)
                def _():
                    wait_in(ring, buf)

                    @plsc.parallel_loop(0, RB, unroll=2)
                    def _rows(r):
                        for u in range(CB // 16):
                            cvec = jnp.int32(u * 16) + lanes
                            colv = ring[buf, r, pl.ds(u * 16, 16)]
                            w = plsc.load_gather(w_tab, [colv])
                            cl = colv - v0
                            m = (cl >= 0) & (cl < VH)
                            cl = jnp.where(m, cl, 0)
                            w = jnp.where(m, w, 0.0)
                            plsc.addupdate_scatter(acc, [cl, cvec], w)

                    @pl.when(chunk + LOOK < NCHUNK)
                    def _():
                        fetch(ring, chunk + LOOK, (buf + LOOK) % NR)

            def grp(j, carry):
                for b in range(NR):
                    step(j * NR + b, b)
                return carry
            lax.fori_loop(0, (NCHUNK + NR - 1) // NR, grp, 0)

            pltpu.sync_copy(
                acc, avg_hbm.at[pl.ds(v0, VH), pl.ds(c0, CB)])
            return carry

        lax.fori_loop(0, 2, walk, 0)

    pl.run_scoped(
        phase_a,
        pltpu.VMEM((VH, CB), jnp.float32),
        pltpu.VMEM((NR, RB, CB), jnp.int32),
    )


@jax.jit
def kernel(indices, w_es):
    run = pl.kernel(
        _body,
        out_type=(
            jax.ShapeDtypeStruct((V, B), jnp.float32),
            jax.ShapeDtypeStruct((V, B), jnp.float32),
        ),
        mesh=plsc.VectorSubcoreMesh(
            core_axis_name="c", subcore_axis_name="s",
            num_cores=2, num_subcores=16,
        ),
        scratch_types=[
            pltpu.VMEM((V,), jnp.float32),
            pltpu.SemaphoreType.DMA((NR,)),
            pltpu.SemaphoreType.DMA((NR,)),
        ],
        compiler_params=pltpu.CompilerParams(
            use_tc_tiling_on_sc=True, needs_layout_passes=False),
    )
    idx_t = jnp.transpose(indices.astype(jnp.int32))
    avg_t, wt = run(idx_t, w_es)
    return jnp.transpose(avg_t), wt


# speedup vs baseline: 451.6003x; 1.2318x over previous
import functools

import jax
import jax.numpy as jnp
from jax import lax
from jax.experimental import pallas as pl
from jax.experimental.pallas import tpu as pltpu
from jax.experimental.pallas import tpu_sc as plsc

B = 4096
V = 1000
CB = 128
RB = 40
NCHUNK = V // RB
NR = 4
LOOK = 2
VH = 504


def _body(idx_hbm, w_hbm, avg_hbm, wt_hbm, w_tab, sem_i, sem_o):
    cid = lax.axis_index("c")
    sid = lax.axis_index("s")
    wid = sid * 2 + cid
    c0 = wid * CB
    lanes = lax.iota(jnp.int32, 16)
    zeros16 = jnp.zeros((16,), jnp.float32)

    pltpu.sync_copy(w_hbm, w_tab)

    def fetch(ring, chunk, buf):
        pltpu.async_copy(
            idx_hbm.at[pl.ds(chunk * RB, RB), pl.ds(c0, CB)],
            ring.at[buf], sem_i.at[buf])

    def wait_in(ring, buf):
        pltpu.make_async_copy(
            idx_hbm.at[pl.ds(0, RB), pl.ds(c0, CB)],
            ring.at[buf], sem_i.at[buf]).wait()

    def phase_w(ring, ring_o):
        def put(chunk, buf):
            pltpu.async_copy(
                ring_o.at[buf],
                wt_hbm.at[pl.ds(chunk * RB, RB), pl.ds(c0, CB)],
                sem_o.at[buf])

        def wait_out(buf):
            pltpu.make_async_copy(
                ring_o.at[buf],
                wt_hbm.at[pl.ds(0, RB), pl.ds(c0, CB)],
                sem_o.at[buf]).wait()

        def step(chunk, buf):
            @pl.when(chunk < NCHUNK)
            def _():
                wait_in(ring, buf)

                @pl.when(chunk >= NR)
                def _():
                    wait_out(buf)

                @plsc.parallel_loop(0, RB, unroll=4)
                def _rows(r):
                    for u in range(CB // 16):
                        colv = ring[buf, r, pl.ds(u * 16, 16)]
                        w = plsc.load_gather(w_tab, [colv])
                        ring_o[buf, r, pl.ds(u * 16, 16)] = w

                put(chunk, buf)

                @pl.when(chunk + LOOK < NCHUNK)
                def _():
                    fetch(ring, chunk + LOOK, (buf + LOOK) % NR)

        for b in range(LOOK):
            fetch(ring, b, b)

        def grp(j, carry):
            for b in range(NR):
                step(j * NR + b, b)
            return carry
        lax.fori_loop(0, (NCHUNK + NR - 1) // NR, grp, 0)
        for b in range(NR):
            wait_out(b)

    pl.run_scoped(
        phase_w,
        pltpu.VMEM((NR, RB, CB), jnp.int32),
        pltpu.VMEM((NR, RB, CB), jnp.float32),
    )

    def phase_a(acc, ring):
        def walk(h, carry):
            v0 = h * (V - VH)

            for b in range(LOOK):
                fetch(ring, b, b)

            @plsc.parallel_loop(0, VH, unroll=4)
            def _zero(row):
                for u in range(CB // 16):
                    acc[row, pl.ds(u * 16, 16)] = zeros16

            def step(chunk, buf):
                @pl.when(chunk < NCHUNK)
                def _():
                    wait_in(ring, buf)

                    @plsc.parallel_loop(0, RB, unroll=4)
                    def _rows(r):
                        for u in range(CB // 16):
                            cvec = jnp.int32(u * 16) + lanes
                            colv = ring[buf, r, pl.ds(u * 16, 16)]
                            w = plsc.load_gather(w_tab, [colv])
                            cl = colv - v0
                            m = (cl >= 0) & (cl < VH)
                            cl = jnp.where(m, cl, 0)
                            w = jnp.where(m, w, 0.0)
                            plsc.addupdate_scatter(acc, [cl, cvec], w)

                    @pl.when(chunk + LOOK < NCHUNK)
                    def _():
                        fetch(ring, chunk + LOOK, (buf + LOOK) % NR)

            def grp(j, carry):
                for b in range(NR):
                    step(j * NR + b, b)
                return carry
            lax.fori_loop(0, (NCHUNK + NR - 1) // NR, grp, 0)

            pltpu.sync_copy(
                acc, avg_hbm.at[pl.ds(v0, VH), pl.ds(c0, CB)])
            return carry

        lax.fori_loop(0, 2, walk, 0)

    pl.run_scoped(
        phase_a,
        pltpu.VMEM((VH, CB), jnp.float32),
        pltpu.VMEM((NR, RB, CB), jnp.int32),
    )


@jax.jit
def kernel(indices, w_es):
    run = pl.kernel(
        _body,
        out_type=(
            jax.ShapeDtypeStruct((V, B), jnp.float32),
            jax.ShapeDtypeStruct((V, B), jnp.float32),
        ),
        mesh=plsc.VectorSubcoreMesh(
            core_axis_name="c", subcore_axis_name="s",
            num_cores=2, num_subcores=16,
        ),
        scratch_types=[
            pltpu.VMEM((V,), jnp.float32),
            pltpu.SemaphoreType.DMA((NR,)),
            pltpu.SemaphoreType.DMA((NR,)),
        ],
        compiler_params=pltpu.CompilerParams(
            use_tc_tiling_on_sc=True, needs_layout_passes=False),
    )
    idx_t = jnp.transpose(indices.astype(jnp.int32))
    avg_t, wt = run(idx_t, w_es)
    return jnp.transpose(avg_t), wt


# speedup vs baseline: 564.7109x; 1.2505x over previous
import jax
import jax.numpy as jnp
from jax import lax
from jax.experimental import pallas as pl
from jax.experimental.pallas import tpu as pltpu
from jax.experimental.pallas import tpu_sc as plsc

B = 4096
V = 1000
CB = 128
RB = 40
NCHUNK = V // RB
NR = 4
LOOK = 2
VH = 504


def _body(idx_hbm, w_hbm, avg_hbm, wt_hbm, w_tab, sem_i, sem_o):
    cid = lax.axis_index("c")
    sid = lax.axis_index("s")
    wid = sid * 2 + cid
    c0 = wid * CB
    lanes = lax.iota(jnp.int32, 16)
    zeros16 = jnp.zeros((16,), jnp.float32)

    pltpu.sync_copy(w_hbm, w_tab)

    def fetch(ring, chunk, buf):
        pltpu.async_copy(
            idx_hbm.at[pl.ds(chunk * RB, RB), pl.ds(c0, CB)],
            ring.at[buf], sem_i.at[buf])

    def wait_in(ring, buf):
        pltpu.make_async_copy(
            idx_hbm.at[pl.ds(0, RB), pl.ds(c0, CB)],
            ring.at[buf], sem_i.at[buf]).wait()

    def phases(acc, ring, ring_o):
        def put(chunk, buf):
            pltpu.async_copy(
                ring_o.at[buf],
                wt_hbm.at[pl.ds(chunk * RB, RB), pl.ds(c0, CB)],
                sem_o.at[buf])

        def wait_out(buf):
            pltpu.make_async_copy(
                ring_o.at[buf],
                wt_hbm.at[pl.ds(0, RB), pl.ds(c0, CB)],
                sem_o.at[buf]).wait()

        def zero_acc():
            @plsc.parallel_loop(0, VH, unroll=4)
            def _zero(row):
                for u in range(CB // 16):
                    acc[row, pl.ds(u * 16, 16)] = zeros16

        def step1(chunk, buf):
            @pl.when(chunk < NCHUNK)
            def _():
                wait_in(ring, buf)

                @pl.when(chunk >= NR)
                def _():
                    wait_out(buf)

                @plsc.parallel_loop(0, RB, unroll=4)
                def _rows(r):
                    for u in range(CB // 16):
                        cvec = jnp.int32(u * 16) + lanes
                        colv = ring[buf, r, pl.ds(u * 16, 16)]
                        w = plsc.load_gather(w_tab, [colv])
                        ring_o[buf, r, pl.ds(u * 16, 16)] = w
                        m = colv < VH
                        cl = jnp.where(m, colv, 0)
                        wm = jnp.where(m, w, 0.0)
                        plsc.addupdate_scatter(acc, [cl, cvec], wm)

                put(chunk, buf)

                @pl.when(chunk + LOOK < NCHUNK)
                def _():
                    fetch(ring, chunk + LOOK, (buf + LOOK) % NR)

        def step2(chunk, buf):
            @pl.when(chunk < NCHUNK)
            def _():
                wait_in(ring, buf)

                @plsc.parallel_loop(0, RB, unroll=4)
                def _rows(r):
                    for u in range(CB // 16):
                        cvec = jnp.int32(u * 16) + lanes
                        colv = ring[buf, r, pl.ds(u * 16, 16)]
                        w = plsc.load_gather(w_tab, [colv])
                        cl = colv - (V - VH)
                        m = cl >= 0
                        cl = jnp.where(m, cl, 0)
                        w = jnp.where(m, w, 0.0)
                        plsc.addupdate_scatter(acc, [cl, cvec], w)

                @pl.when(chunk + LOOK < NCHUNK)
                def _():
                    fetch(ring, chunk + LOOK, (buf + LOOK) % NR)

        for b in range(LOOK):
            fetch(ring, b, b)
        zero_acc()

        def grp1(j, carry):
            for b in range(NR):
                step1(j * NR + b, b)
            return carry
        lax.fori_loop(0, (NCHUNK + NR - 1) // NR, grp1, 0)
        pltpu.sync_copy(acc, avg_hbm.at[pl.ds(0, VH), pl.ds(c0, CB)])
        for b in range(NR):
            wait_out(b)

        for b in range(LOOK):
            fetch(ring, b, b)
        zero_acc()

        def grp2(j, carry):
            for b in range(NR):
                step2(j * NR + b, b)
            return carry
        lax.fori_loop(0, (NCHUNK + NR - 1) // NR, grp2, 0)
        pltpu.sync_copy(acc, avg_hbm.at[pl.ds(V - VH, VH), pl.ds(c0, CB)])

    pl.run_scoped(
        phases,
        pltpu.VMEM((VH, CB), jnp.float32),
        pltpu.VMEM((NR, RB, CB), jnp.int32),
        pltpu.VMEM((NR, RB, CB), jnp.float32),
    )


@jax.jit
def kernel(indices, w_es):
    run = pl.kernel(
        _body,
        out_type=(
            jax.ShapeDtypeStruct((V, B), jnp.float32),
            jax.ShapeDtypeStruct((V, B), jnp.float32),
        ),
        mesh=plsc.VectorSubcoreMesh(
            core_axis_name="c", subcore_axis_name="s",
            num_cores=2, num_subcores=16,
        ),
        scratch_types=[
            pltpu.VMEM((V,), jnp.float32),
            pltpu.SemaphoreType.DMA((NR,)),
            pltpu.SemaphoreType.DMA((NR,)),
        ],
        compiler_params=pltpu.CompilerParams(
            use_tc_tiling_on_sc=True, needs_layout_passes=False),
    )
    idx_t = jnp.transpose(indices.astype(jnp.int32))
    avg_t, wt = run(idx_t, w_es)
    return jnp.transpose(avg_t), wt
